# baseline (speedup 1.0000x reference)
import functools

import jax
import jax.numpy as jnp
from jax.experimental import pallas as pl
from jax.experimental.pallas import tpu as pltpu

N0, N1, N2 = 10000, 5000, 1024
D, H, C = 128, 128, 16
E0, E1 = 160000, 32768

def _seg_kernel(src_ref, dst_ref, w_ref, x_ref, acc_ref, wsum_ref):
    @pl.when(pl.program_id(0) == 0)
    def _init():
        acc_ref[...] = jnp.zeros_like(acc_ref)
        wsum_ref[...] = jnp.zeros_like(wsum_ref)

    def body(e, _):
        src = src_ref[0, 0, e]
        dst = dst_ref[0, 0, e]
        w = w_ref[0, 0, e]
        row = x_ref[pl.ds(src, 1), :]
        acc_ref[pl.ds(dst, 1), :] = acc_ref[pl.ds(dst, 1), :] + w * row
        wsum_ref[pl.ds(dst, 1), :] = wsum_ref[pl.ds(dst, 1), :] + w
        return 0

    jax.lax.fori_loop(0, src_ref.shape[-1], body, 0)


def _segment_sums(x, src, dst, w, num_dst):
    e = src.shape[0]
    chunk = 1000 if e % 1000 == 0 else 1024
    nch = e // chunk
    src3 = src.reshape(nch, 1, chunk)
    dst3 = dst.reshape(nch, 1, chunk)
    w3 = w.reshape(nch, 1, chunk)
    acc, wsum = pl.pallas_call(
        _seg_kernel,
        grid=(nch,),
        in_specs=[
            pl.BlockSpec((1, 1, chunk), lambda i: (i, 0, 0), memory_space=pltpu.SMEM),
            pl.BlockSpec((1, 1, chunk), lambda i: (i, 0, 0), memory_space=pltpu.SMEM),
            pl.BlockSpec((1, 1, chunk), lambda i: (i, 0, 0), memory_space=pltpu.SMEM),
            pl.BlockSpec(x.shape, lambda i: (0, 0)),
        ],
        out_specs=[
            pl.BlockSpec((num_dst, D), lambda i: (0, 0)),
            pl.BlockSpec((num_dst, D), lambda i: (0, 0)),
        ],
        out_shape=[
            jax.ShapeDtypeStruct((num_dst, D), jnp.float32),
            jax.ShapeDtypeStruct((num_dst, D), jnp.float32),
        ],
    )(src3, dst3, w3, x)
    return acc, wsum


def _post1_kernel(acc_ref, wsum_ref, hd_ref, w_ref, b_ref, o_ref):
    hn = (acc_ref[...] + hd_ref[...]) / (wsum_ref[...] + 1.0)
    o_ref[...] = jax.nn.relu(
        jnp.dot(hn, w_ref[...], preferred_element_type=jnp.float32) + b_ref[...]
    )


def _post2_kernel(acc_ref, wsum_ref, hd_ref, w_ref, b_ref, wfc_ref, bfc_ref, o_ref):
    hn = (acc_ref[...] + hd_ref[...]) / (wsum_ref[...] + 1.0)
    h = jax.nn.relu(
        jnp.dot(hn, w_ref[...], preferred_element_type=jnp.float32) + b_ref[...]
    )
    o_ref[...] = jnp.dot(h, wfc_ref[...], preferred_element_type=jnp.float32) + bfc_ref[...]


def kernel(x, edge_index_0, edge_weight_0, edge_index_1, edge_weight_1,
           W0, b0, W1, b1, Wfc, bfc):
    src0, dst0 = edge_index_0[0], edge_index_0[1]
    src1, dst1 = edge_index_1[0], edge_index_1[1]

    acc0, wsum0 = _segment_sums(x, src0, dst0, edge_weight_0, N1)
    h1 = pl.pallas_call(
        _post1_kernel,
        out_shape=jax.ShapeDtypeStruct((N1, H), jnp.float32),
    )(acc0, wsum0, x[:N1], W0, b0.reshape(1, H))

    acc1, wsum1 = _segment_sums(h1, src1, dst1, edge_weight_1, N2)
    out = pl.pallas_call(
        _post2_kernel,
        out_shape=jax.ShapeDtypeStruct((N2, C), jnp.float32),
    )(acc1, wsum1, h1[:N2], W1, b1.reshape(1, H), Wfc, bfc.reshape(1, C))
    return out


# speedup vs baseline: 4.1112x; 4.1112x over previous
import functools

import jax
import jax.numpy as jnp
from jax import lax
from jax.experimental import pallas as pl
from jax.experimental.pallas import tpu as pltpu
from jax.experimental.pallas import tpu_sc as plsc

N0, N1, N2 = 10000, 5000, 1024
D, H, C = 128, 128, 16
E0, E1 = 160000, 32768

_K = 128
_NT = 32
_NSUB = 16


def _sc_seg_body(nch, npad, table_ref, src_ref, dst_ref, w_ref,
                 acc_out, accw_out,
                 src_v, dst_v, w_v, gbuf, wbuf, acc_sh, accw_sh, gsem):
    c = lax.axis_index("c")
    s = lax.axis_index("s")
    wid = c * _NSUB + s
    rpt = npad // _NSUB

    pltpu.sync_copy(src_ref.at[wid], src_v)
    pltpu.sync_copy(dst_ref.at[wid], dst_v)
    pltpu.sync_copy(w_ref.at[wid], w_v)

    zeros16 = jnp.zeros((16,), jnp.float32)

    def zrow(r, carry):
        for f in range(8):
            gbuf[r, pl.ds(16 * f, 16)] = zeros16
            wbuf[r, pl.ds(16 * f, 16)] = zeros16
        return carry

    lax.fori_loop(0, _K, zrow, 0)

    base = s * rpt
    for k in range(rpt // 64):
        pltpu.sync_copy(gbuf.at[pl.ds(0, 64)],
                        acc_sh.at[pl.ds(base + 64 * k, 64)])
        pltpu.sync_copy(wbuf.at[pl.ds(0, 64)],
                        accw_sh.at[pl.ds(base + 64 * k, 64)])
    plsc.subcore_barrier()

    lane0 = lax.iota(jnp.int32, 16) == 0

    def chunk_body(ch, carry):
        pltpu.async_copy(table_ref.at[src_v.at[ch]], gbuf, gsem).wait()

        def grp_body(g, carry2):
            wv16 = w_v[ch, pl.ds(g * 16, 16)]
            for j in range(16):
                r = g * 16 + j
                wvec = jnp.full((16,), wv16[j], jnp.float32)
                for f in range(8):
                    sl = pl.ds(16 * f, 16)
                    gbuf[r, sl] = gbuf[r, sl] * wvec
                wbuf[r, pl.ds(0, 16)] = jnp.where(lane0, wvec, zeros16)
            return carry2

        lax.fori_loop(0, _K // 16, grp_body, 0)
        pltpu.sync_copy(gbuf, acc_sh.at[dst_v.at[ch]], add=True)
        pltpu.sync_copy(wbuf, accw_sh.at[dst_v.at[ch]], add=True)
        return carry

    lax.fori_loop(0, nch, chunk_body, 0)
    plsc.subcore_barrier()

    for k in range(rpt // 64):
        sl = pl.ds(base + 64 * k, 64)
        pltpu.sync_copy(acc_sh.at[sl], acc_out.at[c].at[sl])
        pltpu.sync_copy(accw_sh.at[sl], accw_out.at[c].at[sl])


def _sc_segment_sums(table, src, dst, w, npad):
    e = src.shape[0]
    epad = -(-e // (_NT * _K)) * (_NT * _K)
    if epad != e:
        pz = epad - e
        src = jnp.concatenate([src, jnp.zeros((pz,), src.dtype)])
        dst = jnp.concatenate([dst, jnp.zeros((pz,), dst.dtype)])
        w = jnp.concatenate([w, jnp.zeros((pz,), w.dtype)])
    nch = epad // (_NT * _K)
    srcT = src.reshape(_NT, nch, _K)
    dstT = dst.reshape(_NT, nch, _K)
    wT = w.reshape(_NT, nch, _K)

    mesh = plsc.VectorSubcoreMesh(core_axis_name="c", subcore_axis_name="s",
                                  num_cores=2, num_subcores=_NSUB)
    f = pl.kernel(
        functools.partial(_sc_seg_body, nch, npad),
        out_type=[
            jax.ShapeDtypeStruct((2, npad, D), jnp.float32),
            jax.ShapeDtypeStruct((2, npad, D), jnp.float32),
        ],
        mesh=mesh,
        scratch_types=[
            pltpu.VMEM((nch, _K), jnp.int32),
            pltpu.VMEM((nch, _K), jnp.int32),
            pltpu.VMEM((nch, _K), jnp.float32),
            pltpu.VMEM((_K, D), jnp.float32),
            pltpu.VMEM((_K, D), jnp.float32),
            pltpu.VMEM_SHARED((npad, D), jnp.float32),
            pltpu.VMEM_SHARED((npad, D), jnp.float32),
            pltpu.SemaphoreType.DMA,
        ],
    )
    return f(table, srcT, dstT, wT)


def _post1_kernel(acc_ref, accw_ref, hd_ref, w_ref, b_ref, o_ref):
    acc = acc_ref[0, :N1, :] + acc_ref[1, :N1, :] + hd_ref[...]
    wsum = accw_ref[0, :N1, 0:1] + accw_ref[1, :N1, 0:1]
    hn = acc / (wsum + 1.0)
    o_ref[...] = jax.nn.relu(
        jnp.dot(hn, w_ref[...], preferred_element_type=jnp.float32) + b_ref[...]
    )


def _post2_kernel(acc_ref, accw_ref, hd_ref, w_ref, b_ref, wfc_ref, bfc_ref,
                  o_ref):
    acc = acc_ref[0, :N2, :] + acc_ref[1, :N2, :] + hd_ref[...]
    wsum = accw_ref[0, :N2, 0:1] + accw_ref[1, :N2, 0:1]
    hn = acc / (wsum + 1.0)
    h = jax.nn.relu(
        jnp.dot(hn, w_ref[...], preferred_element_type=jnp.float32) + b_ref[...]
    )
    o_ref[...] = jnp.dot(h, wfc_ref[...],
                         preferred_element_type=jnp.float32) + bfc_ref[...]


def kernel(x, edge_index_0, edge_weight_0, edge_index_1, edge_weight_1,
           W0, b0, W1, b1, Wfc, bfc):
    src0, dst0 = edge_index_0[0], edge_index_0[1]
    src1, dst1 = edge_index_1[0], edge_index_1[1]

    acc0, accw0 = _sc_segment_sums(x[:N1], src0, dst0, edge_weight_0, 5120)
    h1 = pl.pallas_call(
        _post1_kernel,
        out_shape=jax.ShapeDtypeStruct((N1, H), jnp.float32),
    )(acc0, accw0, x[:N1], W0, b0.reshape(1, H))

    acc1, accw1 = _sc_segment_sums(h1, src1, dst1, edge_weight_1, N2)
    out = pl.pallas_call(
        _post2_kernel,
        out_shape=jax.ShapeDtypeStruct((N2, C), jnp.float32),
    )(acc1, accw1, h1[:N2], W1, b1.reshape(1, H), Wfc, bfc.reshape(1, C))
    return out


# speedup vs baseline: 5.2958x; 1.2881x over previous
import functools

import jax
import jax.numpy as jnp
from jax import lax
from jax.experimental import pallas as pl
from jax.experimental.pallas import tpu as pltpu
from jax.experimental.pallas import tpu_sc as plsc

N0, N1, N2 = 10000, 5000, 1024
D, H, C = 128, 128, 16
E0, E1 = 160000, 32768

_K = 128
_NT = 32
_NSUB = 16


def _sc_seg_body(nch, npad, table_ref, src_ref, dst_ref, w_ref,
                 acc_out, wsum_out,
                 src_v, dst_v, w_v, gbufs, wacc, acc_sh, gsems, ssems):
    c = lax.axis_index("c")
    s = lax.axis_index("s")
    wid = c * _NSUB + s
    rpt = npad // _NSUB

    pltpu.sync_copy(src_ref.at[wid], src_v)
    pltpu.sync_copy(dst_ref.at[wid], dst_v)
    pltpu.sync_copy(w_ref.at[wid], w_v)

    zeros16 = jnp.zeros((16,), jnp.float32)

    def zrow(r, carry):
        for f in range(8):
            gbufs[0][r, pl.ds(16 * f, 16)] = zeros16
        return carry

    lax.fori_loop(0, _K, zrow, 0)

    def zw(i, carry):
        wacc[pl.ds(i * 16, 16)] = zeros16
        return carry

    lax.fori_loop(0, npad // 16, zw, 0)

    base = s * rpt
    for k in range(rpt // 64):
        pltpu.sync_copy(gbufs[0].at[pl.ds(0, 64)],
                        acc_sh.at[pl.ds(base + 64 * k, 64)])
    plsc.subcore_barrier()

    iota16 = lax.iota(jnp.int32, 16)
    ng = len(gbufs)

    def gather(ch, gi):
        return pltpu.async_copy(table_ref.at[src_v.at[ch]], gbufs[gi],
                                gsems[gi])

    def scat_acc(ch, gi):
        return pltpu.async_copy(gbufs[gi], acc_sh.at[dst_v.at[ch]],
                                ssems[gi], add=True)

    def wait_gather(ch, gi):
        pltpu.make_async_copy(table_ref.at[src_v.at[ch]], gbufs[gi],
                              gsems[gi]).wait()

    def wait_scat_acc(ch, gi):
        pltpu.make_async_copy(gbufs[gi], acc_sh.at[dst_v.at[ch]],
                              ssems[gi]).wait()

    gather(0, 0)
    gather(1, 1)

    def quad_body(q, carry):
        ch0 = q * ng
        for b in range(ng):
            ch = ch0 + b
            wait_gather(ch, b)

            def grp_body(g, carry2):
                wv16 = w_v[ch, pl.ds(g * 16, 16)]
                dv16 = dst_v[ch, pl.ds(g * 16, 16)]
                base16 = (dv16 >> 4) << 4
                lane16 = dv16 & 15
                for j in range(16):
                    r = g * 16 + j
                    wvec = jnp.full((16,), wv16[j], jnp.float32)
                    for f in range(8):
                        sl = pl.ds(16 * f, 16)
                        gbufs[b][r, sl] = gbufs[b][r, sl] * wvec
                    off = base16[j]
                    hit = iota16 == lane16[j]
                    wacc[pl.ds(off, 16)] = wacc[pl.ds(off, 16)] + jnp.where(
                        hit, wvec, zeros16)
                return carry2

            lax.fori_loop(0, _K // 16, grp_body, 0)
            scat_acc(ch, b)

            nxt = ch + 2
            ni = (b + 2) % ng

            @pl.when(nxt < nch)
            def _():
                @pl.when(nxt >= ng)
                def _():
                    wait_scat_acc(nxt - ng, ni)
                gather(nxt, ni)
        return carry

    lax.fori_loop(0, nch // ng, quad_body, 0)

    for b in range(ng):
        wait_scat_acc(nch - ng + b, b)
    plsc.subcore_barrier()

    for k in range(rpt // 64):
        sl = pl.ds(base + 64 * k, 64)
        pltpu.sync_copy(acc_sh.at[sl], acc_out.at[c].at[sl])
    pltpu.sync_copy(wacc, wsum_out.at[wid])


def _sc_segment_sums(table, src, dst, w, npad):
    e = src.shape[0]
    epad = -(-e // (_NT * _K)) * (_NT * _K)
    if epad != e:
        pz = epad - e
        src = jnp.concatenate([src, jnp.zeros((pz,), src.dtype)])
        dst = jnp.concatenate([dst, jnp.zeros((pz,), dst.dtype)])
        w = jnp.concatenate([w, jnp.zeros((pz,), w.dtype)])
    nch = epad // (_NT * _K)
    srcT = src.reshape(_NT, nch, _K)
    dstT = dst.reshape(_NT, nch, _K)
    wT = w.reshape(_NT, nch, _K)

    mesh = plsc.VectorSubcoreMesh(core_axis_name="c", subcore_axis_name="s",
                                  num_cores=2, num_subcores=_NSUB)
    f = pl.kernel(
        functools.partial(_sc_seg_body, nch, npad),
        out_type=[
            jax.ShapeDtypeStruct((2, npad, D), jnp.float32),
            jax.ShapeDtypeStruct((_NT, npad), jnp.float32),
        ],
        mesh=mesh,
        scratch_types=[
            pltpu.VMEM((nch, _K), jnp.int32),
            pltpu.VMEM((nch, _K), jnp.int32),
            pltpu.VMEM((nch, _K), jnp.float32),
            [pltpu.VMEM((_K, D), jnp.float32) for _ in range(4)],
            pltpu.VMEM((npad,), jnp.float32),
            pltpu.VMEM_SHARED((npad, D), jnp.float32),
            [pltpu.SemaphoreType.DMA for _ in range(4)],
            [pltpu.SemaphoreType.DMA for _ in range(4)],
        ],
    )
    return f(table, srcT, dstT, wT)


def _post1_kernel(acc_ref, accw_ref, hd_ref, w_ref, b_ref, o_ref):
    acc = acc_ref[0, :N1, :] + acc_ref[1, :N1, :] + hd_ref[...]
    wsum = jnp.sum(accw_ref[:, :N1], axis=0)[:, None]
    hn = acc / (wsum + 1.0)
    o_ref[...] = jax.nn.relu(
        jnp.dot(hn, w_ref[...], preferred_element_type=jnp.float32) + b_ref[...]
    )


def _post2_kernel(acc_ref, accw_ref, hd_ref, w_ref, b_ref, wfc_ref, bfc_ref,
                  o_ref):
    acc = acc_ref[0, :N2, :] + acc_ref[1, :N2, :] + hd_ref[...]
    wsum = jnp.sum(accw_ref[:, :N2], axis=0)[:, None]
    hn = acc / (wsum + 1.0)
    h = jax.nn.relu(
        jnp.dot(hn, w_ref[...], preferred_element_type=jnp.float32) + b_ref[...]
    )
    o_ref[...] = jnp.dot(h, wfc_ref[...],
                         preferred_element_type=jnp.float32) + bfc_ref[...]


def kernel(x, edge_index_0, edge_weight_0, edge_index_1, edge_weight_1,
           W0, b0, W1, b1, Wfc, bfc):
    src0, dst0 = edge_index_0[0], edge_index_0[1]
    src1, dst1 = edge_index_1[0], edge_index_1[1]

    acc0, accw0 = _sc_segment_sums(x[:N1], src0, dst0, edge_weight_0, 5120)
    h1 = pl.pallas_call(
        _post1_kernel,
        out_shape=jax.ShapeDtypeStruct((N1, H), jnp.float32),
    )(acc0, accw0, x[:N1], W0, b0.reshape(1, H))

    acc1, accw1 = _sc_segment_sums(h1, src1, dst1, edge_weight_1, N2)
    out = pl.pallas_call(
        _post2_kernel,
        out_shape=jax.ShapeDtypeStruct((N2, C), jnp.float32),
    )(acc1, accw1, h1[:N2], W1, b1.reshape(1, H), Wfc, bfc.reshape(1, C))
    return out
